# baseline (speedup 1.0000x reference)
import functools

import jax
import jax.numpy as jnp
from jax import lax
from jax.experimental import pallas as pl
from jax.experimental.pallas import tpu as pltpu
from jax.experimental.pallas import tpu_sc as plsc

EMBED = 2048
EXPERTS = 64
K = 8
BLOCK = 2048
NTOK = 16384
NW = 32
TPW = NTOK // NW
L = 16
GROUPS = TPW // L


def _scores_body(x1_ref, x2_ref, w_ref, st_ref):
    w = w_ref[...]
    l1 = jax.lax.dot_general(
        x1_ref[...], w[:, : EMBED // 2], (((1,), (1,)), ((), ())),
        preferred_element_type=jnp.float32,
    )
    l2 = jax.lax.dot_general(
        x2_ref[...], w[:, EMBED // 2 :], (((1,), (1,)), ((), ())),
        preferred_element_type=jnp.float32,
    )
    lt = (l1 + l2).T
    m = jnp.max(lt, axis=0, keepdims=True)
    e = jnp.exp(lt - m)
    s = jnp.sum(e, axis=0, keepdims=True)
    st_ref[...] = e / s


_mesh = plsc.VectorSubcoreMesh(core_axis_name="c", subcore_axis_name="s")


@functools.partial(
    pl.kernel,
    mesh=_mesh,
    out_type=[
        jax.ShapeDtypeStruct((NTOK * K,), jnp.int32),
        jax.ShapeDtypeStruct((NTOK * K,), jnp.float32),
    ],
    scratch_types=[
        pltpu.VMEM((EXPERTS, TPW), jnp.float32),
        pltpu.VMEM((TPW * K,), jnp.int32),
        pltpu.VMEM((TPW * K,), jnp.float32),
    ],
)
def _sc_topk(st_hbm, idx_hbm, wgt_hbm, st_v, idx_v, wgt_v):
    wid = lax.axis_index("s") * 2 + lax.axis_index("c")
    base = wid * TPW
    pltpu.sync_copy(st_hbm.at[:, pl.ds(base, TPW)], st_v)

    def group(g, carry):
        tv = [jnp.full((L,), -1.0, jnp.float32) for _ in range(K)]
        ti = [jnp.full((L,), 0.0, jnp.float32) for _ in range(K)]
        for e in range(EXPERTS):
            v = st_v[e, pl.ds(g * L, L)]
            i = jnp.full((L,), float(e), jnp.float32)
            for j in range(K):
                keep = tv[j] >= v
                nv = jnp.maximum(tv[j], v)
                ni = jnp.where(keep, ti[j], i)
                v, i = jnp.minimum(tv[j], v), jnp.where(keep, i, ti[j])
                tv[j], ti[j] = nv, ni
        for j in range(K):
            off = (g * K + j) * L
            wgt_v[pl.ds(off, L)] = tv[j]
            idx_v[pl.ds(off, L)] = ti[j].astype(jnp.int32)
        return carry

    lax.fori_loop(0, GROUPS, group, 0)
    pltpu.sync_copy(idx_v, idx_hbm.at[pl.ds(base * K, TPW * K)])
    pltpu.sync_copy(wgt_v, wgt_hbm.at[pl.ds(base * K, TPW * K)])


@jax.jit
def kernel(hidden_states, weight):
    x = hidden_states.reshape(-1, EMBED)
    n = x.shape[0]
    grid = n // BLOCK
    st = pl.pallas_call(
        _scores_body,
        grid=(grid,),
        in_specs=[
            pl.BlockSpec((BLOCK, EMBED // 2), lambda i: (i, 0)),
            pl.BlockSpec((BLOCK, EMBED // 2), lambda i: (i, 1)),
            pl.BlockSpec((EXPERTS, EMBED), lambda i: (0, 0)),
        ],
        out_specs=pl.BlockSpec((EXPERTS, BLOCK), lambda i: (0, i)),
        out_shape=jax.ShapeDtypeStruct((EXPERTS, n), jnp.float32),
    )(x, x, weight)
    idxf, wgtf = _sc_topk(st)
    idx = idxf.reshape(n // L, K, L).transpose(0, 2, 1).reshape(n, K)
    wgt = wgtf.reshape(n // L, K, L).transpose(0, 2, 1).reshape(n, K)
    return (idx, wgt)


# speedup vs baseline: 1.6601x; 1.6601x over previous
import jax
import jax.numpy as jnp
from jax.experimental import pallas as pl

EMBED = 2048
HALF = EMBED // 2
QUAD = EMBED // 4
EXPERTS = 64
K = 8
BLOCK = 2048


def _body(x1_ref, x2_ref, x3_ref, x4_ref, w_ref, idx_ref, wgt_ref):
    w = w_ref[...]
    parts = []
    for q, xr in enumerate((x1_ref, x2_ref, x3_ref, x4_ref)):
        parts.append(jax.lax.dot_general(
            xr[...], w[:, q * QUAD : (q + 1) * QUAD], (((1,), (1,)), ((), ())),
            preferred_element_type=jnp.float32,
        ))
    logits = (parts[0] + parts[1]) + (parts[2] + parts[3])
    lt = logits.T
    m = jnp.max(lt, axis=0, keepdims=True)
    e = jnp.exp(lt - m)
    s = jnp.sum(e, axis=0, keepdims=True)
    sc = e / s
    iota = jax.lax.broadcasted_iota(jnp.int32, sc.shape, 0).astype(jnp.float32)
    vals, idxs = [], []
    for _ in range(K):
        mj = jnp.max(sc, axis=0, keepdims=True)
        hit = sc == mj
        ij = jnp.min(jnp.where(hit, iota, float(EXPERTS)), axis=0, keepdims=True)
        vals.append(mj)
        idxs.append(ij)
        sc = jnp.where(iota == ij, -1.0, sc)
    wgt_ref[...] = jnp.concatenate(vals, axis=0).T
    idx_ref[...] = jnp.concatenate(idxs, axis=0).T.astype(jnp.int32)


@jax.jit
def kernel(hidden_states, weight):
    x = hidden_states.reshape(-1, EMBED)
    n = x.shape[0]
    grid = n // BLOCK
    idx, wgt = pl.pallas_call(
        _body,
        grid=(grid,),
        in_specs=[
            pl.BlockSpec((BLOCK, QUAD), lambda i: (i, 0)),
            pl.BlockSpec((BLOCK, QUAD), lambda i: (i, 1)),
            pl.BlockSpec((BLOCK, QUAD), lambda i: (i, 2)),
            pl.BlockSpec((BLOCK, QUAD), lambda i: (i, 3)),
            pl.BlockSpec((EXPERTS, EMBED), lambda i: (0, 0)),
        ],
        out_specs=[
            pl.BlockSpec((BLOCK, K), lambda i: (i, 0)),
            pl.BlockSpec((BLOCK, K), lambda i: (i, 0)),
        ],
        out_shape=[
            jax.ShapeDtypeStruct((n, K), jnp.int32),
            jax.ShapeDtypeStruct((n, K), jnp.float32),
        ],
    )(x, x, x, x, weight)
    return (idx, wgt)


# speedup vs baseline: 1.6621x; 1.0012x over previous
import jax
import jax.numpy as jnp
from jax.experimental import pallas as pl

EMBED = 2048
HALF = EMBED // 2
EXPERTS = 64
K = 8
BLOCK = 2048


def _body(x1_ref, x2_ref, w_ref, idx_ref, wgt_ref):
    w = w_ref[...]
    l1 = jax.lax.dot_general(
        x1_ref[...], w[:, :HALF], (((1,), (1,)), ((), ())),
        preferred_element_type=jnp.float32,
    )
    l2 = jax.lax.dot_general(
        x2_ref[...], w[:, HALF:], (((1,), (1,)), ((), ())),
        preferred_element_type=jnp.float32,
    )
    logits = l1 + l2
    lt = logits.T
    m = jnp.max(lt, axis=0, keepdims=True)
    e = jnp.exp(lt - m)
    s = jnp.sum(e, axis=0, keepdims=True)
    sc = e / s
    iota = jax.lax.broadcasted_iota(jnp.int32, sc.shape, 0).astype(jnp.float32)
    vals, idxs = [], []
    for _ in range(K):
        mj = jnp.max(sc, axis=0, keepdims=True)
        hit = sc == mj
        ij = jnp.min(jnp.where(hit, iota, float(EXPERTS)), axis=0, keepdims=True)
        vals.append(mj)
        idxs.append(ij)
        sc = jnp.where(iota == ij, -1.0, sc)
    wgt_ref[...] = jnp.concatenate(vals, axis=0).T
    idx_ref[...] = jnp.concatenate(idxs, axis=0).T.astype(jnp.int32)


@jax.jit
def kernel(hidden_states, weight):
    x = hidden_states.reshape(-1, EMBED)
    n = x.shape[0]
    grid = n // BLOCK
    idx, wgt = pl.pallas_call(
        _body,
        grid=(grid,),
        in_specs=[
            pl.BlockSpec((BLOCK, HALF), lambda i: (i, 0)),
            pl.BlockSpec((BLOCK, HALF), lambda i: (i, 1)),
            pl.BlockSpec((EXPERTS, EMBED), lambda i: (0, 0)),
        ],
        out_specs=[
            pl.BlockSpec((BLOCK, K), lambda i: (i, 0)),
            pl.BlockSpec((BLOCK, K), lambda i: (i, 0)),
        ],
        out_shape=[
            jax.ShapeDtypeStruct((n, K), jnp.int32),
            jax.ShapeDtypeStruct((n, K), jnp.float32),
        ],
    )(x, x, weight)
    return (idx, wgt)
